# Initial kernel scaffold; baseline (speedup 1.0000x reference)
import jax
import jax.numpy as jnp
from jax.experimental import pallas as pl


def kernel(x, edge_index, W_s2d, b_s2d, W_d2s, b_d2s):
    raise NotImplementedError("write your pallas kernel here")


# speedup vs baseline: 2.4854x; 2.4854x over previous
import jax
import jax.numpy as jnp
from jax import lax
from jax.experimental import pallas as pl
from jax.experimental.pallas import tpu as pltpu
from jax.experimental.pallas import tpu_sc as plsc

N = 10000
NPAD = 10240
E = 160000
EPAD = 163840
D = 256
H = 128
BATCH = 64
NBT = EPAD // 16 // BATCH
CH = 8
NCHK = NBT // CH
NPT = NPAD // 16
DCH = 128
ROUND = 5120
GARB = 32
RPT = ROUND // 16
RCH = 32
BN = 1024


def _mesh():
    return plsc.VectorSubcoreMesh(core_axis_name="c", subcore_axis_name="s")


NH = NPAD + 16


def _deg_body(row2d, col2d, degr_out, degc_out, hists, idxb, hist, sumb, out16):
    cid = lax.axis_index("c")
    tid = lax.axis_index("s")

    def run(idx2d, deg_out, cc):
        def zh(i, _):
            hist[pl.ds(i * 16, 16)] = jnp.zeros((16,), jnp.float32)
            return 0

        lax.fori_loop(0, NH // 16, zh, 0)

        def chunk(c, _):
            pltpu.sync_copy(idx2d.at[pl.ds(tid * NBT + c * CH, CH)], idxb)
            for i in range(CH):
                for v in range(BATCH // 16):
                    xi = idxb[i, pl.ds(v * 16, 16)]
                    cnt, last = plsc.scan_count(xi)
                    plsc.addupdate_scatter(hist, [xi], cnt.astype(jnp.float32),
                                           mask=last)
            return 0

        lax.fori_loop(0, NCHK, chunk, 0)
        pltpu.sync_copy(hist, hists.at[cc, tid])
        plsc.subcore_barrier()

        for c2 in range(NPT // 128):
            nb = tid * NPT + c2 * 128
            pltpu.sync_copy(hists.at[cc, :, pl.ds(nb, 128)], sumb)
            for j in range(8):
                sl = pl.ds(j * 16, 16)
                s = sumb[0, sl]
                for r in range(1, 16):
                    s = s + sumb[r, sl]
                for j2 in range(16):
                    out16[j * 16 + j2, pl.ds(0, 16)] = jnp.full(
                        (16,), s[j2], jnp.float32)
            pltpu.sync_copy(out16, deg_out.at[pl.ds(nb, 128)])

    @pl.when(cid == 0)
    def _():
        run(row2d, degr_out, 0)

    @pl.when(cid == 1)
    def _():
        run(col2d, degc_out, 1)


def _sc_degrees(row2d, col2d):
    fn = pl.kernel(
        _deg_body,
        out_type=[jax.ShapeDtypeStruct((NPAD, 16), jnp.float32)] * 2
        + [jax.ShapeDtypeStruct((2, 16, NH), jnp.float32)],
        mesh=_mesh(),
        scratch_types=[
            pltpu.VMEM((CH, BATCH), jnp.int32),
            pltpu.VMEM((NH,), jnp.float32),
            pltpu.VMEM((16, 128), jnp.float32),
            pltpu.VMEM((128, 16), jnp.float32),
        ],
        compiler_params=pltpu.CompilerParams(needs_layout_passes=False),
    )
    degr16, degc16, _ = fn(row2d, col2d)
    return degr16, degc16


def _tc_body(x_ref, w1_ref, w2_ref, degr_ref, degc_ref, *out_refs):
    z1_refs = out_refs[0:2]
    z2_refs = out_refs[2:4]
    dr_ref, dc_ref = out_refs[4], out_refs[5]
    h = pl.program_id(1)
    xb = x_ref[...]
    degr = degr_ref[...]
    degc = degc_ref[...]
    dr = jnp.where(degr > 0, lax.rsqrt(jnp.maximum(degr, 1e-12)), 0.0)
    dc = jnp.where(degc > 0, lax.rsqrt(jnp.maximum(degc, 1e-12)), 0.0)
    dn = (((1,), (1,)), ((), ()))
    y1 = lax.dot_general(xb, w1_ref[...], dn, preferred_element_type=jnp.float32)
    y2 = lax.dot_general(xb, w2_ref[...], dn, preferred_element_type=jnp.float32)
    z1 = (0.5 * dc[:, 0:1]) * y1
    z2 = (0.5 * dr[:, 0:1]) * y2
    for j in range(2):
        @pl.when(h == j)
        def _(j=j):
            z1_refs[j][...] = z1
            z2_refs[j][...] = z2

    @pl.when(h == 0)
    def _():
        dr_ref[...] = dr
        dc_ref[...] = dc


def _tc_linear(x_pad, w1, w2, degr16, degc16):
    nblk = NPAD // BN
    return pl.pallas_call(
        _tc_body,
        grid=(nblk, 2),
        in_specs=[
            pl.BlockSpec((BN, D), lambda i, h: (i, 0)),
            pl.BlockSpec((H, D), lambda i, h: (h, 0)),
            pl.BlockSpec((H, D), lambda i, h: (h, 0)),
            pl.BlockSpec((BN, 16), lambda i, h: (i, 0)),
            pl.BlockSpec((BN, 16), lambda i, h: (i, 0)),
        ],
        out_specs=[pl.BlockSpec((BN, H), lambda i, h: (i, 0)) for _ in range(4)]
        + [pl.BlockSpec((BN, 16), lambda i, h: (i, 0)) for _ in range(2)],
        out_shape=[jax.ShapeDtypeStruct((NPAD, H), jnp.float32) for _ in range(4)]
        + [jax.ShapeDtypeStruct((NPAD, 16), jnp.float32) for _ in range(2)],
    )(x_pad, w1, w2, degr16, degc16)


def _edge_body(z1h0, z1h1, z2h0, z2h1, row2d, col2d,
               dr16, dc16, bias1d, o0, o1,
               gch, sch, ich, gbuf, d1, ob, drb, dcb, bb, acc):
    cid = lax.axis_index("c")
    tid = lax.axis_index("s")
    z1s = (z1h0, z1h1)
    z2s = (z2h0, z2h1)
    outs = (o0, o1)

    def zero_gbuf():
        def zfill(i, _):
            for v in range(8):
                gbuf[i, pl.ds(v * 16, 16)] = jnp.zeros((16,), jnp.float32)
            return 0

        lax.fori_loop(0, BATCH, zfill, 0)

    def zero_acc():
        zero_gbuf()

        def zk(k, _):
            pltpu.sync_copy(gbuf, acc.at[pl.ds(tid * RPT + k * BATCH, BATCH)])
            return 0

        lax.fori_loop(0, RPT // BATCH, zk, 0)
        plsc.subcore_barrier()

    def edge_pass(z_ref, gather_hbm, scatter_hbm, lo):

        def chunk(c, _):
            pltpu.sync_copy(gather_hbm.at[pl.ds(tid * NBT + c * CH, CH)], gch)
            pltpu.sync_copy(scatter_hbm.at[pl.ds(tid * NBT + c * CH, CH)], sch)
            for i in range(CH):
                for j in range(BATCH // 16):
                    sl = pl.ds(j * 16, 16)
                    d = sch[i, sl] - lo
                    ok = (d >= 0) & (d < ROUND)
                    garb = ROUND + ((lax.iota(jnp.int32, 16) + i + 8 * c) &
                                    jnp.int32(GARB - 1))
                    ich[i, sl] = jnp.where(ok, d, garb)
            for i in range(CH):
                pltpu.sync_copy(z_ref.at[gch.at[i]], gbuf)
                pltpu.sync_copy(gbuf, acc.at[ich.at[i]], add=True)
            return 0

        lax.fori_loop(0, NCHK, chunk, 0)
        plsc.subcore_barrier()

    def run_half(cc):
        o_r = outs[cc]
        pltpu.sync_copy(bias1d.at[pl.ds(cc * H, H)], bb)

        def round_body(r, _):
            lo = r * ROUND

            zero_acc()
            edge_pass(z1s[cc], col2d, row2d, lo)

            def drain1(k, _):
                local = tid * RPT + k * RCH
                base = lo + local
                pltpu.sync_copy(acc.at[pl.ds(local, RCH)], d1)
                pltpu.sync_copy(dr16.at[pl.ds(base, RCH)], drb)

                def drain_i(i, _):
                    a = drb[i, pl.ds(0, 16)][0]
                    for v in range(8):
                        sl = pl.ds(v * 16, 16)
                        ob[i, sl] = a * d1[i, sl] + bb[sl]
                    return 0

                lax.fori_loop(0, RCH, drain_i, 0)
                pltpu.sync_copy(ob, o_r.at[pl.ds(base, RCH)])
                return 0

            lax.fori_loop(0, RPT // RCH, drain1, 0)
            plsc.subcore_barrier()

            zero_acc()
            edge_pass(z2s[cc], row2d, col2d, lo)

            def drain2(k, _):
                local = tid * RPT + k * RCH
                base = lo + local
                pltpu.sync_copy(acc.at[pl.ds(local, RCH)], d1)
                pltpu.sync_copy(o_r.at[pl.ds(base, RCH)], ob)
                pltpu.sync_copy(dc16.at[pl.ds(base, RCH)], dcb)

                def drain_i(i, _):
                    c = dcb[i, pl.ds(0, 16)][0]
                    for v in range(8):
                        sl = pl.ds(v * 16, 16)
                        ob[i, sl] = ob[i, sl] + c * d1[i, sl]
                    return 0

                lax.fori_loop(0, RCH, drain_i, 0)
                pltpu.sync_copy(ob, o_r.at[pl.ds(base, RCH)])
                return 0

            lax.fori_loop(0, RPT // RCH, drain2, 0)
            plsc.subcore_barrier()
            return 0

        lax.fori_loop(0, NPAD // ROUND, round_body, 0)

    for cc in range(2):
        @pl.when(cid == cc)
        def _(cc=cc):
            run_half(cc)


def _sc_edge(z1s, z2s, row2d, col2d, dr16, dc16, bias1d):
    fn = pl.kernel(
        _edge_body,
        out_type=[jax.ShapeDtypeStruct((NPAD, H), jnp.float32)] * 2,
        mesh=_mesh(),
        scratch_types=[
            pltpu.VMEM((CH, BATCH), jnp.int32),
            pltpu.VMEM((CH, BATCH), jnp.int32),
            pltpu.VMEM((CH, BATCH), jnp.int32),
            pltpu.VMEM((BATCH, H), jnp.float32),
            pltpu.VMEM((RCH, H), jnp.float32),
            pltpu.VMEM((RCH, H), jnp.float32),
            pltpu.VMEM((RCH, 16), jnp.float32),
            pltpu.VMEM((RCH, 16), jnp.float32),
            pltpu.VMEM((H,), jnp.float32),
            pltpu.VMEM_SHARED((ROUND + GARB, H), jnp.float32),
        ],
        compiler_params=pltpu.CompilerParams(needs_layout_passes=False),
    )
    return fn(*z1s, *z2s, row2d, col2d, dr16, dc16, bias1d)


def kernel(x, edge_index, W_s2d, b_s2d, W_d2s, b_d2s):
    pad_ids = jnp.full((EPAD - E,), NPAD, jnp.int32)
    row2d = jnp.concatenate([edge_index[0], pad_ids]).reshape(EPAD // BATCH, BATCH)
    col2d = jnp.concatenate([edge_index[1], pad_ids]).reshape(EPAD // BATCH, BATCH)
    x_pad = jnp.pad(x, ((0, NPAD - N), (0, 0)))
    degr16, degc16 = _sc_degrees(row2d, col2d)
    outs = _tc_linear(x_pad, W_s2d, W_d2s, degr16, degc16)
    z1s = [jnp.pad(z, ((0, GARB), (0, 0))) for z in outs[0:2]]
    z2s = [jnp.pad(z, ((0, GARB), (0, 0))) for z in outs[2:4]]
    dr16, dc16 = outs[4], outs[5]
    bias1d = 0.5 * (b_s2d + b_d2s)
    o = _sc_edge(z1s, z2s, row2d, col2d, dr16, dc16, bias1d)
    return jnp.concatenate(o, axis=1)[:N]


# speedup vs baseline: 2.7585x; 1.1099x over previous
import jax
import jax.numpy as jnp
from jax import lax
from jax.experimental import pallas as pl
from jax.experimental.pallas import tpu as pltpu
from jax.experimental.pallas import tpu_sc as plsc

N = 10000
NPAD = 10240
E = 160000
EPAD = 163840
D = 256
H = 128
BATCH = 64
NBT = EPAD // 16 // BATCH
CH = 8
NCHK = NBT // CH
NPT = NPAD // 16
DCH = 128
ROUND = 5120
GARB = 32
RPT = ROUND // 16
RCH = 32
BN = 1024


def _mesh():
    return plsc.VectorSubcoreMesh(core_axis_name="c", subcore_axis_name="s")


NH = NPAD + 16


def _deg_body(row2d, col2d, degr_out, degc_out, hists, idxb, hist, sumb, out16):
    cid = lax.axis_index("c")
    tid = lax.axis_index("s")

    def run(idx2d, deg_out, cc):
        def zh(i, _):
            hist[pl.ds(i * 16, 16)] = jnp.zeros((16,), jnp.float32)
            return 0

        lax.fori_loop(0, NH // 16, zh, 0)

        def chunk(c, _):
            pltpu.sync_copy(idx2d.at[pl.ds(tid * NBT + c * CH, CH)], idxb)
            for i in range(CH):
                for v in range(BATCH // 16):
                    xi = idxb[i, pl.ds(v * 16, 16)]
                    cnt, last = plsc.scan_count(xi)
                    plsc.addupdate_scatter(hist, [xi], cnt.astype(jnp.float32),
                                           mask=last)
            return 0

        lax.fori_loop(0, NCHK, chunk, 0)
        pltpu.sync_copy(hist, hists.at[cc, tid])
        plsc.subcore_barrier()

        for c2 in range(NPT // 128):
            nb = tid * NPT + c2 * 128
            pltpu.sync_copy(hists.at[cc, :, pl.ds(nb, 128)], sumb)
            for j in range(8):
                sl = pl.ds(j * 16, 16)
                s = sumb[0, sl]
                for r in range(1, 16):
                    s = s + sumb[r, sl]
                for j2 in range(16):
                    out16[j * 16 + j2, pl.ds(0, 16)] = jnp.full(
                        (16,), s[j2], jnp.float32)
            pltpu.sync_copy(out16, deg_out.at[pl.ds(nb, 128)])

    @pl.when(cid == 0)
    def _():
        run(row2d, degr_out, 0)

    @pl.when(cid == 1)
    def _():
        run(col2d, degc_out, 1)


def _sc_degrees(row2d, col2d):
    fn = pl.kernel(
        _deg_body,
        out_type=[jax.ShapeDtypeStruct((NPAD, 16), jnp.float32)] * 2
        + [jax.ShapeDtypeStruct((2, 16, NH), jnp.float32)],
        mesh=_mesh(),
        scratch_types=[
            pltpu.VMEM((CH, BATCH), jnp.int32),
            pltpu.VMEM((NH,), jnp.float32),
            pltpu.VMEM((16, 128), jnp.float32),
            pltpu.VMEM((128, 16), jnp.float32),
        ],
        compiler_params=pltpu.CompilerParams(needs_layout_passes=False),
    )
    degr16, degc16, _ = fn(row2d, col2d)
    return degr16, degc16


def _tc_body(x_ref, w1_ref, w2_ref, degr_ref, degc_ref, *out_refs):
    z1_refs = out_refs[0:2]
    z2_refs = out_refs[2:4]
    dr_ref, dc_ref = out_refs[4], out_refs[5]
    h = pl.program_id(1)
    xb = x_ref[...]
    degr = degr_ref[...]
    degc = degc_ref[...]
    dr = jnp.where(degr > 0, lax.rsqrt(jnp.maximum(degr, 1e-12)), 0.0)
    dc = jnp.where(degc > 0, lax.rsqrt(jnp.maximum(degc, 1e-12)), 0.0)
    dn = (((1,), (1,)), ((), ()))
    y1 = lax.dot_general(xb, w1_ref[...], dn, preferred_element_type=jnp.float32)
    y2 = lax.dot_general(xb, w2_ref[...], dn, preferred_element_type=jnp.float32)
    z1 = (0.5 * dc[:, 0:1]) * y1
    z2 = (0.5 * dr[:, 0:1]) * y2
    for j in range(2):
        @pl.when(h == j)
        def _(j=j):
            z1_refs[j][...] = z1
            z2_refs[j][...] = z2

    @pl.when(h == 0)
    def _():
        dr_ref[...] = dr
        dc_ref[...] = dc


def _tc_linear(x_pad, w1, w2, degr16, degc16):
    nblk = NPAD // BN
    return pl.pallas_call(
        _tc_body,
        grid=(nblk, 2),
        in_specs=[
            pl.BlockSpec((BN, D), lambda i, h: (i, 0)),
            pl.BlockSpec((H, D), lambda i, h: (h, 0)),
            pl.BlockSpec((H, D), lambda i, h: (h, 0)),
            pl.BlockSpec((BN, 16), lambda i, h: (i, 0)),
            pl.BlockSpec((BN, 16), lambda i, h: (i, 0)),
        ],
        out_specs=[pl.BlockSpec((BN, H), lambda i, h: (i, 0)) for _ in range(4)]
        + [pl.BlockSpec((BN, 16), lambda i, h: (i, 0)) for _ in range(2)],
        out_shape=[jax.ShapeDtypeStruct((NPAD, H), jnp.float32) for _ in range(4)]
        + [jax.ShapeDtypeStruct((NPAD, 16), jnp.float32) for _ in range(2)],
    )(x_pad, w1, w2, degr16, degc16)


def _edge_body(z1h0, z1h1, z2h0, z2h1, row2d, col2d,
               dr16, dc16, bias1d, o0, o1,
               gch, sch, ich, gbuf, gbuf2, semA, semB, d1, ob, drb, dcb, bb,
               acc):
    cid = lax.axis_index("c")
    tid = lax.axis_index("s")
    z1s = (z1h0, z1h1)
    z2s = (z2h0, z2h1)
    outs = (o0, o1)

    def zero_gbuf():
        def zfill(i, _):
            for v in range(8):
                gbuf[i, pl.ds(v * 16, 16)] = jnp.zeros((16,), jnp.float32)
            return 0

        lax.fori_loop(0, BATCH, zfill, 0)

    def zero_acc():
        zero_gbuf()

        def zk(k, _):
            pltpu.sync_copy(gbuf, acc.at[pl.ds(tid * RPT + k * BATCH, BATCH)])
            return 0

        lax.fori_loop(0, RPT // BATCH, zk, 0)
        plsc.subcore_barrier()

    def edge_pass(z_ref, gather_hbm, scatter_hbm, lo):

        bufs = (gbuf, gbuf2)
        sems = (semA, semB)

        def chunk(c, _):
            pltpu.sync_copy(gather_hbm.at[pl.ds(tid * NBT + c * CH, CH)], gch)
            pltpu.sync_copy(scatter_hbm.at[pl.ds(tid * NBT + c * CH, CH)], sch)
            for i in range(CH):
                for j in range(BATCH // 16):
                    sl = pl.ds(j * 16, 16)
                    d = sch[i, sl] - lo
                    ok = (d >= 0) & (d < ROUND)
                    garb = ROUND + ((lax.iota(jnp.int32, 16) + i + 8 * c) &
                                    jnp.int32(GARB - 1))
                    ich[i, sl] = jnp.where(ok, d, garb)
            descs = {0: pltpu.async_copy(z_ref.at[gch.at[0]], bufs[0], sems[0])}
            for i in range(CH):
                descs[i].wait()
                if i + 1 < CH:
                    descs[i + 1] = pltpu.async_copy(
                        z_ref.at[gch.at[i + 1]], bufs[(i + 1) % 2],
                        sems[(i + 1) % 2])
                pltpu.sync_copy(bufs[i % 2], acc.at[ich.at[i]], add=True)
            return 0

        lax.fori_loop(0, NCHK, chunk, 0)
        plsc.subcore_barrier()

    def run_half(cc):
        o_r = outs[cc]
        pltpu.sync_copy(bias1d.at[pl.ds(cc * H, H)], bb)

        def round_body(r, _):
            lo = r * ROUND

            zero_acc()
            edge_pass(z1s[cc], col2d, row2d, lo)

            def drain1(k, _):
                local = tid * RPT + k * RCH
                base = lo + local
                pltpu.sync_copy(acc.at[pl.ds(local, RCH)], d1)
                pltpu.sync_copy(dr16.at[pl.ds(base, RCH)], drb)

                def drain_i(i, _):
                    a = drb[i, pl.ds(0, 16)][0]
                    for v in range(8):
                        sl = pl.ds(v * 16, 16)
                        ob[i, sl] = a * d1[i, sl] + bb[sl]
                    return 0

                lax.fori_loop(0, RCH, drain_i, 0)
                pltpu.sync_copy(ob, o_r.at[pl.ds(base, RCH)])
                return 0

            lax.fori_loop(0, RPT // RCH, drain1, 0)
            plsc.subcore_barrier()

            zero_acc()
            edge_pass(z2s[cc], row2d, col2d, lo)

            def drain2(k, _):
                local = tid * RPT + k * RCH
                base = lo + local
                pltpu.sync_copy(acc.at[pl.ds(local, RCH)], d1)
                pltpu.sync_copy(o_r.at[pl.ds(base, RCH)], ob)
                pltpu.sync_copy(dc16.at[pl.ds(base, RCH)], dcb)

                def drain_i(i, _):
                    c = dcb[i, pl.ds(0, 16)][0]
                    for v in range(8):
                        sl = pl.ds(v * 16, 16)
                        ob[i, sl] = ob[i, sl] + c * d1[i, sl]
                    return 0

                lax.fori_loop(0, RCH, drain_i, 0)
                pltpu.sync_copy(ob, o_r.at[pl.ds(base, RCH)])
                return 0

            lax.fori_loop(0, RPT // RCH, drain2, 0)
            plsc.subcore_barrier()
            return 0

        lax.fori_loop(0, NPAD // ROUND, round_body, 0)

    for cc in range(2):
        @pl.when(cid == cc)
        def _(cc=cc):
            run_half(cc)


def _sc_edge(z1s, z2s, row2d, col2d, dr16, dc16, bias1d):
    fn = pl.kernel(
        _edge_body,
        out_type=[jax.ShapeDtypeStruct((NPAD, H), jnp.float32)] * 2,
        mesh=_mesh(),
        scratch_types=[
            pltpu.VMEM((CH, BATCH), jnp.int32),
            pltpu.VMEM((CH, BATCH), jnp.int32),
            pltpu.VMEM((CH, BATCH), jnp.int32),
            pltpu.VMEM((BATCH, H), jnp.float32),
            pltpu.VMEM((BATCH, H), jnp.float32),
            pltpu.SemaphoreType.DMA,
            pltpu.SemaphoreType.DMA,
            pltpu.VMEM((RCH, H), jnp.float32),
            pltpu.VMEM((RCH, H), jnp.float32),
            pltpu.VMEM((RCH, 16), jnp.float32),
            pltpu.VMEM((RCH, 16), jnp.float32),
            pltpu.VMEM((H,), jnp.float32),
            pltpu.VMEM_SHARED((ROUND + GARB, H), jnp.float32),
        ],
        compiler_params=pltpu.CompilerParams(needs_layout_passes=False),
    )
    return fn(*z1s, *z2s, row2d, col2d, dr16, dc16, bias1d)


def kernel(x, edge_index, W_s2d, b_s2d, W_d2s, b_d2s):
    pad_ids = jnp.full((EPAD - E,), NPAD, jnp.int32)
    row2d = jnp.concatenate([edge_index[0], pad_ids]).reshape(EPAD // BATCH, BATCH)
    col2d = jnp.concatenate([edge_index[1], pad_ids]).reshape(EPAD // BATCH, BATCH)
    x_pad = jnp.pad(x, ((0, NPAD - N), (0, 0)))
    degr16, degc16 = _sc_degrees(row2d, col2d)
    outs = _tc_linear(x_pad, W_s2d, W_d2s, degr16, degc16)
    z1s = [jnp.pad(z, ((0, GARB), (0, 0))) for z in outs[0:2]]
    z2s = [jnp.pad(z, ((0, GARB), (0, 0))) for z in outs[2:4]]
    dr16, dc16 = outs[4], outs[5]
    bias1d = 0.5 * (b_s2d + b_d2s)
    o = _sc_edge(z1s, z2s, row2d, col2d, dr16, dc16, bias1d)
    return jnp.concatenate(o, axis=1)[:N]
